# baseline (speedup 1.0000x reference)
import jax
import jax.numpy as jnp
from jax.experimental import pallas as pl

NUM_NEIGHBORS = 16
OUT_CHANNELS = 32


def _smlp(x, W, g, b, eps=1e-5):
    y = jnp.einsum("...c,cd->...d", x, W)
    axes = tuple(range(y.ndim - 1))
    mean = y.mean(axis=axes)
    var = y.var(axis=axes)
    y = (y - mean) / jnp.sqrt(var + eps) * g + b
    return jax.nn.relu(y)


def _final_stage_kernel(x_ref, w_ref, g_ref, b_ref, o_ref):
    x = x_ref[...]
    w = w_ref[...]
    y = jnp.dot(x, w, preferred_element_type=jnp.float32)
    mean = jnp.mean(y, axis=0, keepdims=True)
    var = jnp.mean((y - mean) ** 2, axis=0, keepdims=True)
    y = (y - mean) / jnp.sqrt(var + 1e-5) * g_ref[...] + b_ref[...]
    o_ref[...] = jnp.maximum(y, 0.0)


def kernel(pts, W1a, g1a, b1a, W1b, g1b, b1b, W2, g2, b2, W3, g3, b3):
    Bb, Nn, _ = pts.shape
    K = NUM_NEIGHBORS
    sq = jnp.sum(pts * pts, axis=-1)
    dist2 = sq[:, :, None] + sq[:, None, :] - 2.0 * jnp.einsum(
        "bnc,bmc->bnm", pts, pts)
    _, knn_idx = jax.lax.top_k(-dist2, K + 1)
    knn_pts = jax.vmap(lambda p, idx: p[idx])(pts, knn_idx)
    abs_pts = knn_pts[:, :, :1, :]
    rel_nbs = knn_pts[:, :, 1:, :] - abs_pts
    dists = jnp.sqrt(jnp.sum(rel_nbs**2, axis=-1, keepdims=True) + 1e-8)
    concat = jnp.concatenate(
        (jnp.broadcast_to(abs_pts, (Bb, Nn, K, 3)), rel_nbs, dists), axis=-1)
    h = concat.reshape(Bb * Nn, K, 7)
    h = _smlp(h, W1a, g1a, b1a)
    h = _smlp(h, W1b, g1b, b1b)
    nbs_pooled = h.reshape(Bb, Nn, K, -1).max(axis=2)
    pts_lifted = _smlp(pts, W2, g2, b2)
    x = jnp.concatenate((pts_lifted, nbs_pooled), axis=-1).reshape(Bb * Nn, 2 * OUT_CHANNELS)
    out = pl.pallas_call(
        _final_stage_kernel,
        out_shape=jax.ShapeDtypeStruct((Bb * Nn, OUT_CHANNELS), jnp.float32),
    )(x, W3, g3.reshape(1, -1), b3.reshape(1, -1))
    return out.reshape(Bb, Nn, OUT_CHANNELS)


# speedup vs baseline: 3.0984x; 3.0984x over previous
import functools

import jax
import jax.numpy as jnp
from jax.experimental import pallas as pl
from jax.experimental.pallas import tpu as pltpu

NUM_NEIGHBORS = 16
OUT_CHANNELS = 32
KP1 = NUM_NEIGHBORS + 1
R = 256
BIG = 3.0e38


def _knn_kernel(rows_ref, ptsT_ref, ptsP_ref, out_ref, scratch_ref):
    rows = rows_ref[0]
    ptsT = ptsT_ref[0]
    ptsP = ptsP_ref[0]
    n = ptsT.shape[1]
    sqr = jnp.sum(rows * rows, axis=1, keepdims=True)
    sqc = jnp.sum(ptsT * ptsT, axis=0, keepdims=True)
    dot = jnp.dot(rows, ptsT, preferred_element_type=jnp.float32)
    dist = (sqr + sqc) - 2.0 * dot
    iota = jax.lax.broadcasted_iota(jnp.int32, (R, n), 1)

    def body(k, d):
        minv = jnp.min(d, axis=1, keepdims=True)
        hit = d == minv
        idxv = jnp.min(jnp.where(hit, iota, n), axis=1, keepdims=True)
        sel = iota == idxv
        coords = jax.lax.dot_general(
            sel.astype(jnp.float32), ptsP,
            (((1,), (0,)), ((), ())),
            precision=jax.lax.Precision.HIGHEST,
            preferred_element_type=jnp.float32)
        scratch_ref[k] = coords
        return jnp.where(sel, BIG, d)

    jax.lax.fori_loop(0, KP1, body, dist, unroll=False)
    out_ref[0] = scratch_ref[...]


def _knn_coords(pts):
    Bb, Nn, _ = pts.shape
    pts_pad = jnp.pad(pts, ((0, 0), (0, 0), (0, 5)))
    ptsT = jnp.transpose(pts_pad, (0, 2, 1))
    nb = Nn // R
    return pl.pallas_call(
        _knn_kernel,
        grid=(Bb, nb),
        in_specs=[
            pl.BlockSpec((1, R, 8), lambda b, i: (b, i, 0)),
            pl.BlockSpec((1, 8, Nn), lambda b, i: (b, 0, 0)),
            pl.BlockSpec((1, Nn, 8), lambda b, i: (b, 0, 0)),
        ],
        out_specs=pl.BlockSpec((1, KP1, R, 8), lambda b, i: (b * nb + i, 0, 0, 0)),
        out_shape=jax.ShapeDtypeStruct((Bb * nb, KP1, R, 8), jnp.float32),
        scratch_shapes=[pltpu.VMEM((KP1, R, 8), jnp.float32)],
    )(pts_pad, ptsT, pts_pad)


def _smlp(x, W, g, b, eps=1e-5):
    y = jnp.einsum("...c,cd->...d", x, W)
    axes = tuple(range(y.ndim - 1))
    mean = y.mean(axis=axes)
    var = y.var(axis=axes)
    y = (y - mean) / jnp.sqrt(var + eps) * g + b
    return jax.nn.relu(y)


def kernel(pts, W1a, g1a, b1a, W1b, g1b, b1b, W2, g2, b2, W3, g3, b3):
    Bb, Nn, _ = pts.shape
    K = NUM_NEIGHBORS
    coords = _knn_coords(pts)
    coords = coords.reshape(Bb, Nn // R, KP1, R, 8)
    knn_pts = jnp.transpose(coords[..., :3], (0, 1, 3, 2, 4)).reshape(
        Bb, Nn, KP1, 3)
    abs_pts = knn_pts[:, :, :1, :]
    rel_nbs = knn_pts[:, :, 1:, :] - abs_pts
    dists = jnp.sqrt(jnp.sum(rel_nbs**2, axis=-1, keepdims=True) + 1e-8)
    concat = jnp.concatenate(
        (jnp.broadcast_to(abs_pts, (Bb, Nn, K, 3)), rel_nbs, dists), axis=-1)
    h = concat.reshape(Bb * Nn, K, 7)
    h = _smlp(h, W1a, g1a, b1a)
    h = _smlp(h, W1b, g1b, b1b)
    nbs_pooled = h.reshape(Bb, Nn, K, -1).max(axis=2)
    pts_lifted = _smlp(pts, W2, g2, b2)
    pts_ebd = _smlp(jnp.concatenate((pts_lifted, nbs_pooled), axis=-1),
                    W3, g3, b3)
    return pts_ebd
